# baseline (speedup 1.0000x reference)
import functools

import jax
import jax.numpy as jnp
import numpy as np
from jax import lax
from jax.experimental import pallas as pl
from jax.experimental.pallas import tpu as pltpu
from jax.experimental.pallas import tpu_sc as plsc

N = 100000
E = 3200000
SUPPORT = 0.05
DELTA = 0.1
C0 = float(10.0 * np.sqrt(2.0 * 9.81 * 0.3))
EPS = SUPPORT * SUPPORT * 0.1
REST_DENSITY = 1000.0
KGRAD_C = float(7.0 / (np.pi * SUPPORT * SUPPORT))
SCALE = float(2.0 * SUPPORT * DELTA * C0)
EPS_LIM = 0.0001 * SUPPORT

NW = 32
EP = 3276800
CH = 2048
NCHT = 100
NG = CH // 16
NP = 100352
NT = NP // 16
NR, NL = 196, 512

_MESH = dict(core_axis_name="c", subcore_axis_name="s", num_cores=2,
             num_subcores=16)


def _ebuf_types():
  return [pltpu.VMEM((CH,), jnp.int32), pltpu.VMEM((CH,), jnp.int32),
          pltpu.VMEM((CH,), jnp.float32), pltpu.VMEM((CH,), jnp.float32),
          pltpu.VMEM((CH,), jnp.float32)]


def _gradw(dxv, dyv, radv):
  q = jnp.clip(radv, 0.0, 1.0)
  omq = 1.0 - q
  f = (-20.0 * KGRAD_C / SUPPORT) * q * omq * omq * omq
  return f * dxv, f * dyv


def _hi16(vf):
  u = lax.bitcast_convert_type(vf, jnp.int32)
  return lax.bitcast_convert_type(u & jnp.int32(-65536), jnp.float32)


def _lo16(vf):
  u = lax.bitcast_convert_type(vf, jnp.int32)
  return lax.bitcast_convert_type(u << 16, jnp.float32)


def _work(c, s, n0, n1):
  nch = jnp.where(c == 0, n0, n1)
  base = c * (16 * n0 * CH) + s * (nch * CH)
  return nch, base


def _zero_accs(z_hbm, tmp, accs, s):
  ns = pl.ds(s * NT, NT)
  for a in accs:
    pltpu.sync_copy(z_hbm.at[ns], tmp)
    pltpu.sync_copy(tmp, a.at[ns])


def _stage_tables(pairs, tmp, s):
  ns = pl.ds(s * NT, NT)
  for hbm, sp in pairs:
    pltpu.sync_copy(hbm.at[ns], tmp)
    pltpu.sync_copy(tmp, sp.at[ns])


def _write_out(accs, tmp, out_hbm, c, s):
  plsc.subcore_barrier()
  ns = pl.ds(s * NT, NT)
  for k, a in enumerate(accs):
    pltpu.sync_copy(a.at[ns], tmp)
    pltpu.sync_copy(tmp, out_hbm.at[c, k, ns])


def _gather_descs(specs, eb, gb, sem):
  return [pltpu.make_async_copy(tbl.at[eb[side]], gb[k], sem)
          for k, (tbl, side) in enumerate(specs)]


def _pipeline(edge_refs, ebufs, gspecs, gbufs, gsems, lsem,
              compute, scatter, nch, ebase):
  def load_linear(q, eb1):
    es = pl.ds(eb1, CH)
    return [pltpu.make_async_copy(r.at[es], b, lsem)
            for r, b in zip(edge_refs, ebufs[q])]

  for d in load_linear(0, ebase):
    d.start()
  for d in load_linear(0, ebase):
    d.wait()
  for d in _gather_descs(gspecs, ebufs[0], gbufs[0], gsems[0]):
    d.start()

  def phase(u, p):
    q = 1 - p
    u1 = u + 1
    eb1 = jnp.where(u1 == nch, ebase, ebase + u1 * CH)
    lds = load_linear(q, eb1)
    for d in lds:
      d.start()
    for d in lds:
      d.wait()
    for d in _gather_descs(gspecs, ebufs[q], gbufs[q], gsems[q]):
      d.start()
    for d in _gather_descs(gspecs, ebufs[p], gbufs[p], gsems[p]):
      d.wait()
    compute(ebufs[p], gbufs[p])
    scatter(ebufs[p])

  @pl.loop(0, nch // 2)
  def _pair(k):
    phase(2 * k, 0)
    phase(2 * k + 1, 1)

  for d in _gather_descs(gspecs, ebufs[0], gbufs[0], gsems[0]):
    d.wait()


def _sc_pass_a(iE, jE, dxE, dyE, radE, volP, zN):
  mesh = plsc.VectorSubcoreMesh(**_MESH)

  @functools.partial(
      pl.kernel,
      out_type=jax.ShapeDtypeStruct((2, 4, NP), jnp.float32),
      mesh=mesh,
      scratch_types=[
          *[pltpu.VMEM_SHARED((NP,), jnp.float32) for _ in range(3)],
          pltpu.VMEM_SHARED((NP,), jnp.float32),
          *_ebuf_types(), *_ebuf_types(),
          *[pltpu.VMEM((CH,), jnp.float32) for _ in range(2)],
          *[pltpu.VMEM((CH,), jnp.float32) for _ in range(3)],
          pltpu.VMEM((NT,), jnp.float32),
          pltpu.SemaphoreType.DMA, pltpu.SemaphoreType.DMA,
          pltpu.SemaphoreType.DMA, pltpu.SemaphoreType.DMA,
      ],
  )
  def body(i_hbm, j_hbm, dx_hbm, dy_hbm, rad_hbm, vol_hbm, z_hbm, out_hbm,
           a0, a1, a2, vol_sp,
           bi0, bj0, bdx0, bdy0, brad0, bi1, bj1, bdx1, bdy1, brad1,
           bvol0, bvol1, v0, v1, v2, tmp,
           lsem, gsemA, gsemB, ssem):
    c = lax.axis_index("c")
    s = lax.axis_index("s")
    nch, ebase = _work(c, s, 32, 68)
    accs = (a0, a1, a2)
    vals = (v0, v1, v2)
    _zero_accs(z_hbm, tmp, accs, s)
    _stage_tables([(vol_hbm, vol_sp)], tmp, s)
    plsc.subcore_barrier()

    def compute(eb, gb):
      bdx, bdy, brad = eb[2], eb[3], eb[4]
      bvol = gb[0]

      @pl.loop(0, NG)
      def _grp(g):
        sl = pl.ds(g * 16, 16)
        dxv = bdx[sl]
        dyv = bdy[sl]
        radv = brad[sl]
        gwx, gwy = _gradw(dxv, dyv, radv)
        rbx = -dxv * radv * SUPPORT
        rby = -dyv * radv * SUPPORT
        fac = bvol[sl] * 2.0
        v0[sl] = rbx * gwx * fac
        v1[sl] = rbx * gwy * fac
        v2[sl] = rby * gwy * fac

    def scatter(eb):
      sds = [pltpu.async_copy(vals[k], accs[k].at[eb[0]], ssem, add=True)
             for k in range(3)]
      for d in sds:
        d.wait()

    _pipeline((i_hbm, j_hbm, dx_hbm, dy_hbm, rad_hbm),
              [(bi0, bj0, bdx0, bdy0, brad0), (bi1, bj1, bdx1, bdy1, brad1)],
              [(vol_sp, 1)], [[bvol0], [bvol1]], (gsemA, gsemB), lsem,
              compute, scatter, nch, ebase)

    _write_out(accs, tmp, out_hbm, c, s)

  return body(iE, jE, dxE, dyE, radE, volP, zN)


def _sc_pass_c(iE, jE, dxE, dyE, radE, li0P, li1P, li2P, rvP, zN):
  mesh = plsc.VectorSubcoreMesh(**_MESH)

  @functools.partial(
      pl.kernel,
      out_type=jax.ShapeDtypeStruct((2, 2, NP), jnp.float32),
      mesh=mesh,
      scratch_types=[
          *[pltpu.VMEM_SHARED((NP,), jnp.float32) for _ in range(2)],
          *[pltpu.VMEM_SHARED((NP,), jnp.float32) for _ in range(4)],
          *_ebuf_types(), *_ebuf_types(),
          *[pltpu.VMEM((CH,), jnp.float32) for _ in range(10)],
          *[pltpu.VMEM((CH,), jnp.float32) for _ in range(2)],
          pltpu.VMEM((NT,), jnp.float32),
          pltpu.SemaphoreType.DMA, pltpu.SemaphoreType.DMA,
          pltpu.SemaphoreType.DMA, pltpu.SemaphoreType.DMA,
      ],
  )
  def body(i_hbm, j_hbm, dx_hbm, dy_hbm, rad_hbm, l0_hbm, l1_hbm, l2_hbm,
           rv_hbm, z_hbm, out_hbm,
           ax, ay, l0_sp, l1_sp, l2_sp, rv_sp,
           bi0, bj0, bdx0, bdy0, brad0, bi1, bj1, bdx1, bdy1, brad1,
           g00, g01, g02, g03, g04,
           g10, g11, g12, g13, g14,
           vx, vy, tmp, lsem, gsemA, gsemB, ssem):
    c = lax.axis_index("c")
    s = lax.axis_index("s")
    nch, ebase = _work(c, s, 56, 44)
    accs = (ax, ay)
    _zero_accs(z_hbm, tmp, accs, s)
    _stage_tables([(l0_hbm, l0_sp), (l1_hbm, l1_sp), (l2_hbm, l2_sp),
                   (rv_hbm, rv_sp)], tmp, s)
    plsc.subcore_barrier()

    gspecs = [(l0_sp, 0), (l1_sp, 0), (l2_sp, 0),
              (rv_sp, 0), (rv_sp, 1)]

    def compute(eb, gb):
      bdx, bdy, brad = eb[2], eb[3], eb[4]
      bl0, bl1, bl2, brvi, brvj = gb

      @pl.loop(0, NG)
      def _grp(g):
        sl = pl.ds(g * 16, 16)
        dxv = bdx[sl]
        dyv = bdy[sl]
        radv = brad[sl]
        gwx, gwy = _gradw(dxv, dyv, radv)
        l01 = bl1[sl]
        ngx = bl0[sl] * gwx + l01 * gwy
        ngy = l01 * gwx + bl2[sl] * gwy
        dwij_mag = jnp.abs(gwx) + jnp.abs(gwy)
        norm_mag = jnp.abs(ngx) + jnp.abs(ngy)
        change = jnp.abs(norm_mag - dwij_mag) / (dwij_mag + EPS_LIM)
        sel = change < 0.1
        gx = jnp.where(sel, ngx, gwx)
        gy = jnp.where(sel, ngy, gwy)
        fac2 = (_hi16(brvj[sl]) - _hi16(brvi[sl])) * _lo16(brvj[sl]) * 2.0
        vx[sl] = fac2 * gx
        vy[sl] = fac2 * gy

    def scatter(eb):
      sds = [pltpu.async_copy(vx, ax.at[eb[0]], ssem, add=True),
             pltpu.async_copy(vy, ay.at[eb[0]], ssem, add=True)]
      for d in sds:
        d.wait()

    _pipeline((i_hbm, j_hbm, dx_hbm, dy_hbm, rad_hbm),
              [(bi0, bj0, bdx0, bdy0, brad0), (bi1, bj1, bdx1, bdy1, brad1)],
              gspecs, [[g00, g01, g02, g03, g04],
                       [g10, g11, g12, g13, g14]],
              (gsemA, gsemB), lsem, compute, scatter, nch, ebase)

    _write_out(accs, tmp, out_hbm, c, s)

  return body(iE, jE, dxE, dyE, radE, li0P, li1P, li2P, rvP, zN)


def _sc_pass_e(iE, jE, dxE, dyE, radE, gxyP, rvP, zN):
  mesh = plsc.VectorSubcoreMesh(**_MESH)

  @functools.partial(
      pl.kernel,
      out_type=jax.ShapeDtypeStruct((2, 1, NP), jnp.float32),
      mesh=mesh,
      scratch_types=[
          pltpu.VMEM_SHARED((NP,), jnp.float32),
          *[pltpu.VMEM_SHARED((NP,), jnp.float32) for _ in range(2)],
          *_ebuf_types(), *_ebuf_types(),
          *[pltpu.VMEM((CH,), jnp.float32) for _ in range(8)],
          pltpu.VMEM((CH,), jnp.float32),
          pltpu.VMEM((NT,), jnp.float32),
          pltpu.SemaphoreType.DMA, pltpu.SemaphoreType.DMA,
          pltpu.SemaphoreType.DMA, pltpu.SemaphoreType.DMA,
      ],
  )
  def body(i_hbm, j_hbm, dx_hbm, dy_hbm, rad_hbm, gxy_hbm, rv_hbm,
           z_hbm, out_hbm,
           acc, gxy_sp, rv_sp,
           bi0, bj0, bdx0, bdy0, brad0, bi1, bj1, bdx1, bdy1, brad1,
           g00, g01, g02, g03,
           g10, g11, g12, g13,
           vv, tmp, lsem, gsemA, gsemB, ssem):
    c = lax.axis_index("c")
    s = lax.axis_index("s")
    nch, ebase = _work(c, s, 56, 44)
    _zero_accs(z_hbm, tmp, (acc,), s)
    _stage_tables([(gxy_hbm, gxy_sp), (rv_hbm, rv_sp)], tmp, s)
    plsc.subcore_barrier()

    gspecs = [(gxy_sp, 0), (rv_sp, 0), (gxy_sp, 1), (rv_sp, 1)]

    def compute(eb, gb):
      bdx, bdy, brad = eb[2], eb[3], eb[4]
      bgxyi, brvi, bgxyj, brvj = gb

      @pl.loop(0, NG)
      def _grp(g):
        sl = pl.ds(g * 16, 16)
        dxv = bdx[sl]
        dyv = bdy[sl]
        radv = brad[sl]
        gwx, gwy = _gradw(dxv, dyv, radv)
        rbx = -dxv * radv * SUPPORT
        rby = -dyv * radv * SUPPORT
        rji2 = rbx * rbx + rby * rby + EPS
        gxyi = bgxyi[sl]
        gxyj = bgxyj[sl]
        density_term = 0.5 * ((_hi16(gxyi) + _hi16(gxyj)) * rbx +
                              (_lo16(gxyi) + _lo16(gxyj)) * rby)
        diffusion_term = _hi16(brvj[sl]) - _hi16(brvi[sl])
        grad_term = (gwx * rbx + gwy * rby) / rji2
        prod = (diffusion_term + density_term) * grad_term
        vv[sl] = prod * _lo16(brvj[sl])

    def scatter(eb):
      pltpu.async_copy(vv, acc.at[eb[0]], ssem, add=True).wait()

    _pipeline((i_hbm, j_hbm, dx_hbm, dy_hbm, rad_hbm),
              [(bi0, bj0, bdx0, bdy0, brad0), (bi1, bj1, bdx1, bdy1, brad1)],
              gspecs, [[g00, g01, g02, g03],
                       [g10, g11, g12, g13]],
              (gsemA, gsemB), lsem, compute, scatter, nch, ebase)

    _write_out((acc,), tmp, out_hbm, c, s)

  return body(iE, jE, dxE, dyE, radE, gxyP, rvP, zN)


def _pack2(x, y):
  xb = lax.bitcast_convert_type(x.astype(jnp.bfloat16), jnp.uint16)
  yb = lax.bitcast_convert_type(y.astype(jnp.bfloat16), jnp.uint16)
  packed = (xb.astype(jnp.uint32) << 16) | yb.astype(jnp.uint32)
  return lax.bitcast_convert_type(packed, jnp.float32)


def _tc_pinv_body(m_ref, dens, vol, li0, li1, li2, rv):
  a = m_ref[0, 0] + m_ref[1, 0]
  b = m_ref[0, 1] + m_ref[1, 1]
  d = m_ref[0, 2] + m_ref[1, 2]
  det = a * d - b * b
  frob2 = a * a + 2.0 * (b * b) + d * d
  use = jnp.abs(det) > 1e-6 * frob2
  sdet = jnp.where(use, det, 1.0)
  sfro = jnp.maximum(frob2, 1e-30)
  li0[...] = jnp.where(use, d / sdet, a / sfro)
  li1[...] = jnp.where(use, -b / sdet, b / sfro)
  li2[...] = jnp.where(use, a / sdet, d / sfro)
  rv[...] = _pack2(dens[...] * REST_DENSITY, vol[...])


def _tc_combine2_body(g_ref, gxy):
  gxy[...] = _pack2(g_ref[0, 0] + g_ref[1, 0], g_ref[0, 1] + g_ref[1, 1])


def _tc_final_body(dp, out):
  out[...] = SCALE * (dp[0, 0] + dp[1, 0])


def kernel(fluidPosition, fluidVolume, fluidDistances, fluidRadialDistances,
           fluidDensity, i, j):
  del fluidPosition
  i = i.astype(jnp.int32)
  j = j.astype(jnp.int32)
  pad = EP - E
  iE = jnp.pad(i, (0, pad))
  jE = jnp.pad(j, (0, pad))
  dxE = jnp.pad(fluidDistances[:, 0], (0, pad))
  dyE = jnp.pad(fluidDistances[:, 1], (0, pad))
  radE = jnp.pad(fluidRadialDistances, (0, pad))
  volP = jnp.pad(fluidVolume, (0, NP - N))
  densP = jnp.pad(fluidDensity, (0, NP - N))
  zN = jnp.zeros((NP,), jnp.float32)
  f32 = jnp.float32
  shp = jax.ShapeDtypeStruct((NR, NL), f32)

  Mpart = _sc_pass_a(iE, jE, dxE, dyE, radE, volP, zN)

  li0, li1, li2, rv2 = pl.pallas_call(
      _tc_pinv_body, out_shape=[shp] * 4)(
          Mpart.reshape(2, 4, NR, NL), densP.reshape(NR, NL),
          volP.reshape(NR, NL))

  Gpart = _sc_pass_c(iE, jE, dxE, dyE, radE,
                     li0.reshape(NP), li1.reshape(NP), li2.reshape(NP),
                     rv2.reshape(NP), zN)

  gxy2 = pl.pallas_call(_tc_combine2_body, out_shape=shp)(
      Gpart.reshape(2, 2, NR, NL))

  Dpart = _sc_pass_e(iE, jE, dxE, dyE, radE,
                     gxy2.reshape(NP), rv2.reshape(NP), zN)

  out2 = pl.pallas_call(_tc_final_body, out_shape=shp)(
      Dpart.reshape(2, 1, NR, NL))
  return out2.reshape(NP)[:N]


# speedup vs baseline: 1.0694x; 1.0694x over previous
import functools

import jax
import jax.numpy as jnp
import numpy as np
from jax import lax
from jax.experimental import pallas as pl
from jax.experimental.pallas import tpu as pltpu
from jax.experimental.pallas import tpu_sc as plsc

N = 100000
E = 3200000
SUPPORT = 0.05
DELTA = 0.1
C0 = float(10.0 * np.sqrt(2.0 * 9.81 * 0.3))
EPS = SUPPORT * SUPPORT * 0.1
REST_DENSITY = 1000.0
KGRAD_C = float(7.0 / (np.pi * SUPPORT * SUPPORT))
SCALE = float(2.0 * SUPPORT * DELTA * C0)
EPS_LIM = 0.0001 * SUPPORT

NW = 32
EP = 3276800
CH = 2048
NCHT = 100
NG = CH // 16
NP = 100352
NT = NP // 16
NR, NL = 196, 512

_MESH = dict(core_axis_name="c", subcore_axis_name="s", num_cores=2,
             num_subcores=16)


def _ebuf_types():
  return [pltpu.VMEM((CH,), jnp.int32), pltpu.VMEM((CH,), jnp.int32),
          pltpu.VMEM((CH,), jnp.float32), pltpu.VMEM((CH,), jnp.float32),
          pltpu.VMEM((CH,), jnp.float32)]


def _gradw(dxv, dyv, radv):
  q = jnp.clip(radv, 0.0, 1.0)
  omq = 1.0 - q
  f = (-20.0 * KGRAD_C / SUPPORT) * q * omq * omq * omq
  return f * dxv, f * dyv


def _hi16(vf):
  u = lax.bitcast_convert_type(vf, jnp.int32)
  return lax.bitcast_convert_type(u & jnp.int32(-65536), jnp.float32)


def _lo16(vf):
  u = lax.bitcast_convert_type(vf, jnp.int32)
  return lax.bitcast_convert_type(u << 16, jnp.float32)


def _work(c, s, n0, n1):
  nch = jnp.where(c == 0, n0, n1)
  base = c * (16 * n0 * CH) + s * (nch * CH)
  return nch, base


def _zero_accs(z_hbm, tmp, accs, s):
  ns = pl.ds(s * NT, NT)
  for a in accs:
    pltpu.sync_copy(z_hbm.at[ns], tmp)
    pltpu.sync_copy(tmp, a.at[ns])


def _stage_tables(pairs, tmp, s):
  ns = pl.ds(s * NT, NT)
  for hbm, sp in pairs:
    pltpu.sync_copy(hbm.at[ns], tmp)
    pltpu.sync_copy(tmp, sp.at[ns])


def _write_out(accs, tmp, out_hbm, c, s):
  plsc.subcore_barrier()
  ns = pl.ds(s * NT, NT)
  for k, a in enumerate(accs):
    pltpu.sync_copy(a.at[ns], tmp)
    pltpu.sync_copy(tmp, out_hbm.at[c, k, ns])


def _gather_descs(specs, eb, gb, sem):
  return [pltpu.make_async_copy(tbl.at[eb[side]], gb[k], sem)
          for k, (tbl, side) in enumerate(specs)]


def _pipeline(edge_refs, ebufs, gspecs, gbufs, gsems, lsem,
              compute, scatter, nch, ebase):
  def load_linear(q, eb1):
    es = pl.ds(eb1, CH)
    return [pltpu.make_async_copy(r.at[es], b, lsem)
            for r, b in zip(edge_refs, ebufs[q])]

  for d in load_linear(0, ebase):
    d.start()
  for d in load_linear(0, ebase):
    d.wait()
  for d in _gather_descs(gspecs, ebufs[0], gbufs[0], gsems[0]):
    d.start()

  def phase(u, p):
    q = 1 - p
    u1 = u + 1
    eb1 = jnp.where(u1 == nch, ebase, ebase + u1 * CH)
    lds = load_linear(q, eb1)
    for d in lds:
      d.start()
    for d in lds:
      d.wait()
    for d in _gather_descs(gspecs, ebufs[q], gbufs[q], gsems[q]):
      d.start()
    for d in _gather_descs(gspecs, ebufs[p], gbufs[p], gsems[p]):
      d.wait()
    compute(ebufs[p], gbufs[p])
    scatter(ebufs[p])

  @pl.loop(0, nch // 2)
  def _pair(k):
    phase(2 * k, 0)
    phase(2 * k + 1, 1)

  for d in _gather_descs(gspecs, ebufs[0], gbufs[0], gsems[0]):
    d.wait()


def _sc_pass_a(iE, jE, dxE, dyE, radE, volP, zN):
  mesh = plsc.VectorSubcoreMesh(**_MESH)

  @functools.partial(
      pl.kernel,
      out_type=jax.ShapeDtypeStruct((2, 4, NP), jnp.float32),
      mesh=mesh,
      scratch_types=[
          *[pltpu.VMEM_SHARED((NP,), jnp.float32) for _ in range(3)],
          pltpu.VMEM_SHARED((NP,), jnp.float32),
          *_ebuf_types(), *_ebuf_types(),
          *[pltpu.VMEM((CH,), jnp.float32) for _ in range(2)],
          *[pltpu.VMEM((CH,), jnp.float32) for _ in range(3)],
          pltpu.VMEM((NT,), jnp.float32),
          pltpu.SemaphoreType.DMA, pltpu.SemaphoreType.DMA,
          pltpu.SemaphoreType.DMA, pltpu.SemaphoreType.DMA,
      ],
  )
  def body(i_hbm, j_hbm, dx_hbm, dy_hbm, rad_hbm, vol_hbm, z_hbm, out_hbm,
           a0, a1, a2, vol_sp,
           bi0, bj0, bdx0, bdy0, brad0, bi1, bj1, bdx1, bdy1, brad1,
           bvol0, bvol1, v0, v1, v2, tmp,
           lsem, gsemA, gsemB, ssem):
    c = lax.axis_index("c")
    s = lax.axis_index("s")
    nch, ebase = _work(c, s, 56, 44)
    accs = (a0, a1, a2)
    vals = (v0, v1, v2)
    _zero_accs(z_hbm, tmp, accs, s)
    _stage_tables([(vol_hbm, vol_sp)], tmp, s)
    plsc.subcore_barrier()

    def compute(eb, gb):
      bdx, bdy, brad = eb[2], eb[3], eb[4]
      bvol = gb[0]

      @pl.loop(0, NG)
      def _grp(g):
        sl = pl.ds(g * 16, 16)
        dxv = bdx[sl]
        dyv = bdy[sl]
        radv = brad[sl]
        gwx, gwy = _gradw(dxv, dyv, radv)
        rbx = -dxv * radv * SUPPORT
        rby = -dyv * radv * SUPPORT
        fac = bvol[sl] * 2.0
        v0[sl] = rbx * gwx * fac
        v1[sl] = rbx * gwy * fac
        v2[sl] = rby * gwy * fac

    def scatter(eb):
      sds = [pltpu.async_copy(vals[k], accs[k].at[eb[0]], ssem, add=True)
             for k in range(3)]
      for d in sds:
        d.wait()

    _pipeline((i_hbm, j_hbm, dx_hbm, dy_hbm, rad_hbm),
              [(bi0, bj0, bdx0, bdy0, brad0), (bi1, bj1, bdx1, bdy1, brad1)],
              [(vol_sp, 1)], [[bvol0], [bvol1]], (gsemA, gsemB), lsem,
              compute, scatter, nch, ebase)

    _write_out(accs, tmp, out_hbm, c, s)

  return body(iE, jE, dxE, dyE, radE, volP, zN)


def _sc_pass_c(iE, jE, dxE, dyE, radE, li0P, li1P, li2P, rvP, zN):
  mesh = plsc.VectorSubcoreMesh(**_MESH)

  @functools.partial(
      pl.kernel,
      out_type=jax.ShapeDtypeStruct((2, 2, NP), jnp.float32),
      mesh=mesh,
      scratch_types=[
          *[pltpu.VMEM_SHARED((NP,), jnp.float32) for _ in range(2)],
          *[pltpu.VMEM_SHARED((NP,), jnp.float32) for _ in range(4)],
          *_ebuf_types(), *_ebuf_types(),
          *[pltpu.VMEM((CH,), jnp.float32) for _ in range(10)],
          *[pltpu.VMEM((CH,), jnp.float32) for _ in range(2)],
          pltpu.VMEM((NT,), jnp.float32),
          pltpu.SemaphoreType.DMA, pltpu.SemaphoreType.DMA,
          pltpu.SemaphoreType.DMA, pltpu.SemaphoreType.DMA,
      ],
  )
  def body(i_hbm, j_hbm, dx_hbm, dy_hbm, rad_hbm, l0_hbm, l1_hbm, l2_hbm,
           rv_hbm, z_hbm, out_hbm,
           ax, ay, l0_sp, l1_sp, l2_sp, rv_sp,
           bi0, bj0, bdx0, bdy0, brad0, bi1, bj1, bdx1, bdy1, brad1,
           g00, g01, g02, g03, g04,
           g10, g11, g12, g13, g14,
           vx, vy, tmp, lsem, gsemA, gsemB, ssem):
    c = lax.axis_index("c")
    s = lax.axis_index("s")
    nch, ebase = _work(c, s, 56, 44)
    accs = (ax, ay)
    _zero_accs(z_hbm, tmp, accs, s)
    _stage_tables([(l0_hbm, l0_sp), (l1_hbm, l1_sp), (l2_hbm, l2_sp),
                   (rv_hbm, rv_sp)], tmp, s)
    plsc.subcore_barrier()

    gspecs = [(l0_sp, 0), (l1_sp, 0), (l2_sp, 0),
              (rv_sp, 0), (rv_sp, 1)]

    def compute(eb, gb):
      bdx, bdy, brad = eb[2], eb[3], eb[4]
      bl0, bl1, bl2, brvi, brvj = gb

      @pl.loop(0, NG)
      def _grp(g):
        sl = pl.ds(g * 16, 16)
        dxv = bdx[sl]
        dyv = bdy[sl]
        radv = brad[sl]
        gwx, gwy = _gradw(dxv, dyv, radv)
        l01 = bl1[sl]
        ngx = bl0[sl] * gwx + l01 * gwy
        ngy = l01 * gwx + bl2[sl] * gwy
        dwij_mag = jnp.abs(gwx) + jnp.abs(gwy)
        norm_mag = jnp.abs(ngx) + jnp.abs(ngy)
        change = jnp.abs(norm_mag - dwij_mag) / (dwij_mag + EPS_LIM)
        sel = change < 0.1
        gx = jnp.where(sel, ngx, gwx)
        gy = jnp.where(sel, ngy, gwy)
        fac2 = (_hi16(brvj[sl]) - _hi16(brvi[sl])) * _lo16(brvj[sl]) * 2.0
        vx[sl] = fac2 * gx
        vy[sl] = fac2 * gy

    def scatter(eb):
      sds = [pltpu.async_copy(vx, ax.at[eb[0]], ssem, add=True),
             pltpu.async_copy(vy, ay.at[eb[0]], ssem, add=True)]
      for d in sds:
        d.wait()

    _pipeline((i_hbm, j_hbm, dx_hbm, dy_hbm, rad_hbm),
              [(bi0, bj0, bdx0, bdy0, brad0), (bi1, bj1, bdx1, bdy1, brad1)],
              gspecs, [[g00, g01, g02, g03, g04],
                       [g10, g11, g12, g13, g14]],
              (gsemA, gsemB), lsem, compute, scatter, nch, ebase)

    _write_out(accs, tmp, out_hbm, c, s)

  return body(iE, jE, dxE, dyE, radE, li0P, li1P, li2P, rvP, zN)


def _sc_pass_e(iE, jE, dxE, dyE, radE, gxyP, rvP, zN):
  mesh = plsc.VectorSubcoreMesh(**_MESH)

  @functools.partial(
      pl.kernel,
      out_type=jax.ShapeDtypeStruct((2, 1, NP), jnp.float32),
      mesh=mesh,
      scratch_types=[
          pltpu.VMEM_SHARED((NP,), jnp.float32),
          *[pltpu.VMEM_SHARED((NP,), jnp.float32) for _ in range(2)],
          *_ebuf_types(), *_ebuf_types(),
          *[pltpu.VMEM((CH,), jnp.float32) for _ in range(8)],
          pltpu.VMEM((CH,), jnp.float32),
          pltpu.VMEM((NT,), jnp.float32),
          pltpu.SemaphoreType.DMA, pltpu.SemaphoreType.DMA,
          pltpu.SemaphoreType.DMA, pltpu.SemaphoreType.DMA,
      ],
  )
  def body(i_hbm, j_hbm, dx_hbm, dy_hbm, rad_hbm, gxy_hbm, rv_hbm,
           z_hbm, out_hbm,
           acc, gxy_sp, rv_sp,
           bi0, bj0, bdx0, bdy0, brad0, bi1, bj1, bdx1, bdy1, brad1,
           g00, g01, g02, g03,
           g10, g11, g12, g13,
           vv, tmp, lsem, gsemA, gsemB, ssem):
    c = lax.axis_index("c")
    s = lax.axis_index("s")
    nch, ebase = _work(c, s, 56, 44)
    _zero_accs(z_hbm, tmp, (acc,), s)
    _stage_tables([(gxy_hbm, gxy_sp), (rv_hbm, rv_sp)], tmp, s)
    plsc.subcore_barrier()

    gspecs = [(gxy_sp, 0), (rv_sp, 0), (gxy_sp, 1), (rv_sp, 1)]

    def compute(eb, gb):
      bdx, bdy, brad = eb[2], eb[3], eb[4]
      bgxyi, brvi, bgxyj, brvj = gb

      @pl.loop(0, NG)
      def _grp(g):
        sl = pl.ds(g * 16, 16)
        dxv = bdx[sl]
        dyv = bdy[sl]
        radv = brad[sl]
        gwx, gwy = _gradw(dxv, dyv, radv)
        rbx = -dxv * radv * SUPPORT
        rby = -dyv * radv * SUPPORT
        rji2 = rbx * rbx + rby * rby + EPS
        gxyi = bgxyi[sl]
        gxyj = bgxyj[sl]
        density_term = 0.5 * ((_hi16(gxyi) + _hi16(gxyj)) * rbx +
                              (_lo16(gxyi) + _lo16(gxyj)) * rby)
        diffusion_term = _hi16(brvj[sl]) - _hi16(brvi[sl])
        grad_term = (gwx * rbx + gwy * rby) / rji2
        prod = (diffusion_term + density_term) * grad_term
        vv[sl] = prod * _lo16(brvj[sl])

    def scatter(eb):
      pltpu.async_copy(vv, acc.at[eb[0]], ssem, add=True).wait()

    _pipeline((i_hbm, j_hbm, dx_hbm, dy_hbm, rad_hbm),
              [(bi0, bj0, bdx0, bdy0, brad0), (bi1, bj1, bdx1, bdy1, brad1)],
              gspecs, [[g00, g01, g02, g03],
                       [g10, g11, g12, g13]],
              (gsemA, gsemB), lsem, compute, scatter, nch, ebase)

    _write_out((acc,), tmp, out_hbm, c, s)

  return body(iE, jE, dxE, dyE, radE, gxyP, rvP, zN)


def _pack2(x, y):
  xb = lax.bitcast_convert_type(x.astype(jnp.bfloat16), jnp.uint16)
  yb = lax.bitcast_convert_type(y.astype(jnp.bfloat16), jnp.uint16)
  packed = (xb.astype(jnp.uint32) << 16) | yb.astype(jnp.uint32)
  return lax.bitcast_convert_type(packed, jnp.float32)


def _tc_pinv_body(m_ref, dens, vol, li0, li1, li2, rv):
  a = m_ref[0, 0] + m_ref[1, 0]
  b = m_ref[0, 1] + m_ref[1, 1]
  d = m_ref[0, 2] + m_ref[1, 2]
  det = a * d - b * b
  frob2 = a * a + 2.0 * (b * b) + d * d
  use = jnp.abs(det) > 1e-6 * frob2
  sdet = jnp.where(use, det, 1.0)
  sfro = jnp.maximum(frob2, 1e-30)
  li0[...] = jnp.where(use, d / sdet, a / sfro)
  li1[...] = jnp.where(use, -b / sdet, b / sfro)
  li2[...] = jnp.where(use, a / sdet, d / sfro)
  rv[...] = _pack2(dens[...] * REST_DENSITY, vol[...])


def _tc_combine2_body(g_ref, gxy):
  gxy[...] = _pack2(g_ref[0, 0] + g_ref[1, 0], g_ref[0, 1] + g_ref[1, 1])


def _tc_final_body(dp, out):
  out[...] = SCALE * (dp[0, 0] + dp[1, 0])


def kernel(fluidPosition, fluidVolume, fluidDistances, fluidRadialDistances,
           fluidDensity, i, j):
  del fluidPosition
  i = i.astype(jnp.int32)
  j = j.astype(jnp.int32)
  pad = EP - E
  iE = jnp.pad(i, (0, pad))
  jE = jnp.pad(j, (0, pad))
  dxE = jnp.pad(fluidDistances[:, 0], (0, pad))
  dyE = jnp.pad(fluidDistances[:, 1], (0, pad))
  radE = jnp.pad(fluidRadialDistances, (0, pad))
  volP = jnp.pad(fluidVolume, (0, NP - N))
  densP = jnp.pad(fluidDensity, (0, NP - N))
  zN = jnp.zeros((NP,), jnp.float32)
  f32 = jnp.float32
  shp = jax.ShapeDtypeStruct((NR, NL), f32)

  Mpart = _sc_pass_a(iE, jE, dxE, dyE, radE, volP, zN)

  li0, li1, li2, rv2 = pl.pallas_call(
      _tc_pinv_body, out_shape=[shp] * 4)(
          Mpart.reshape(2, 4, NR, NL), densP.reshape(NR, NL),
          volP.reshape(NR, NL))

  Gpart = _sc_pass_c(iE, jE, dxE, dyE, radE,
                     li0.reshape(NP), li1.reshape(NP), li2.reshape(NP),
                     rv2.reshape(NP), zN)

  gxy2 = pl.pallas_call(_tc_combine2_body, out_shape=shp)(
      Gpart.reshape(2, 2, NR, NL))

  Dpart = _sc_pass_e(iE, jE, dxE, dyE, radE,
                     gxy2.reshape(NP), rv2.reshape(NP), zN)

  out2 = pl.pallas_call(_tc_final_body, out_shape=shp)(
      Dpart.reshape(2, 1, NR, NL))
  return out2.reshape(NP)[:N]
